# Initial kernel scaffold; baseline (speedup 1.0000x reference)
import jax
import jax.numpy as jnp
from jax.experimental import pallas as pl


def kernel(items, weight):
    raise NotImplementedError("write your pallas kernel here")


# speedup vs baseline: 10.1017x; 10.1017x over previous
import functools

import jax
import jax.numpy as jnp
from jax import lax
from jax.experimental import pallas as pl
from jax.experimental.pallas import tpu as pltpu
from jax.experimental.pallas import tpu_sc as plsc

VOCAB = 100000
EMBED_DIM = 128
BATCH = 16384
HIST = 200

NUM_CORES = 2
NUM_SUBCORES = 16
LANES = 16
NUM_WORKERS = NUM_CORES * NUM_SUBCORES
ROWS_PER_WORKER = BATCH // NUM_WORKERS
RC = 32
NUM_CHUNKS = ROWS_PER_WORKER // RC
HALF = HIST // 2
D_VREGS = EMBED_DIM // LANES


def _sc_body(items_hbm, weight_hbm, out_hbm, idx_v, rows_v, out_v, sem):
    wid = lax.axis_index("s") * NUM_CORES + lax.axis_index("c")
    base = wid * ROWS_PER_WORKER
    inv_n = jnp.float32(1.0 / HIST)

    def chunk_body(c, carry):
        row0 = base + c * RC
        pltpu.sync_copy(items_hbm.at[pl.ds(row0, RC)], idx_v)

        def row_body(r, carry2):
            cp0 = pltpu.async_copy(weight_hbm.at[idx_v.at[r, 0]],
                                   rows_v.at[0], sem)
            cp1 = pltpu.async_copy(weight_hbm.at[idx_v.at[r, 1]],
                                   rows_v.at[1], sem)
            cp0.wait()
            cp1.wait()

            def red(j, acc):
                return tuple(
                    acc[h * D_VREGS + d]
                    + rows_v[h, j, pl.ds(d * LANES, LANES)]
                    for h in range(2) for d in range(D_VREGS)
                )

            acc0 = tuple(jnp.zeros((LANES,), jnp.float32)
                         for _ in range(2 * D_VREGS))
            acc = lax.fori_loop(0, HALF, red, acc0)
            for d in range(D_VREGS):
                out_v[r, pl.ds(d * LANES, LANES)] = (
                    (acc[d] + acc[D_VREGS + d]) * inv_n)
            return carry2

        lax.fori_loop(0, RC, row_body, 0)
        pltpu.sync_copy(out_v, out_hbm.at[pl.ds(row0, RC)])
        return carry

    lax.fori_loop(0, NUM_CHUNKS, chunk_body, 0)


@jax.jit
def kernel(items, weight):
    items32 = items.astype(jnp.int32).reshape(BATCH, 2, HALF)
    mesh = plsc.VectorSubcoreMesh(
        core_axis_name="c", subcore_axis_name="s",
        num_cores=NUM_CORES, num_subcores=NUM_SUBCORES)
    k = pl.kernel(
        _sc_body,
        out_type=jax.ShapeDtypeStruct((BATCH, EMBED_DIM), jnp.float32),
        mesh=mesh,
        scratch_types=[
            pltpu.VMEM((RC, 2, HALF), jnp.int32),
            pltpu.VMEM((2, HALF, EMBED_DIM), jnp.float32),
            pltpu.VMEM((RC, EMBED_DIM), jnp.float32),
            pltpu.SemaphoreType.DMA,
        ],
    )
    return k(items32, weight)
